# Initial kernel scaffold; baseline (speedup 1.0000x reference)
import jax
import jax.numpy as jnp
from jax.experimental import pallas as pl


def kernel(x, pos, edge_index):
    raise NotImplementedError("write your pallas kernel here")


# speedup vs baseline: 122.8413x; 122.8413x over previous
import functools

import jax
import jax.numpy as jnp
from jax import lax
from jax.experimental import pallas as pl
from jax.experimental.pallas import tpu as pltpu
from jax.experimental.pallas import tpu_sc as plsc

DSIGMA_DT = -0.0001
PHI_CUT = 0.3
EPS = 1e-8

NC = 2
NS = 16
NW = NC * NS
CHUNK = 128
SCH = 8
LANES = 16


def _iota16():
  return lax.iota(jnp.int32, LANES)


def _c16(v, dtype=jnp.int32):
  return jnp.full((LANES,), v, dtype)


def _edge_compute(sbuf, dbuf, contrib):
  iot = _iota16()
  c0 = _c16(0)
  c1 = _c16(1)
  c2 = _c16(2)
  c3 = _c16(3)
  for q in range(CHUNK // LANES):
    row = iot + q * LANES
    sx = plsc.load_gather(sbuf, [row, c0])
    sy = plsc.load_gather(sbuf, [row, c1])
    sz = plsc.load_gather(sbuf, [row, c2])
    st = plsc.load_gather(sbuf, [row, c3])
    dx = plsc.load_gather(dbuf, [row, c0])
    dy = plsc.load_gather(dbuf, [row, c1])
    dz = plsc.load_gather(dbuf, [row, c2])
    dt = plsc.load_gather(dbuf, [row, c3])
    ex = sx - dx
    ey = sy - dy
    ez = sz - dz
    et = st - dt
    rr = ex * ex + ey * ey + ez * ez + jnp.float32(EPS)
    w = et / rr
    plsc.store_scatter(contrib, [row, c0], w * ex)
    plsc.store_scatter(contrib, [row, c1], w * ey)
    plsc.store_scatter(contrib, [row, c2], w * ez)


_SC_PARAMS = pltpu.CompilerParams(
    needs_layout_passes=False, use_tc_tiling_on_sc=False)


def _make_main(n_pad, g_per_w):
  mesh = plsc.VectorSubcoreMesh(core_axis_name="c", subcore_axis_name="s")
  rows_t = n_pad // NS

  @functools.partial(
      pl.kernel,
      mesh=mesh,
      compiler_params=_SC_PARAMS,
      out_type=jax.ShapeDtypeStruct((NC, n_pad, 4), jnp.float32),
      scratch_types=[
          pltpu.VMEM((SCH, CHUNK), jnp.int32),
          pltpu.VMEM((SCH, CHUNK), jnp.int32),
          pltpu.VMEM((CHUNK, 4), jnp.float32),
          pltpu.VMEM((CHUNK, 4), jnp.float32),
          pltpu.VMEM((CHUNK, 4), jnp.float32),
          pltpu.VMEM((CHUNK, 4), jnp.float32),
          pltpu.VMEM((CHUNK, 4), jnp.float32),
          pltpu.VMEM_SHARED((n_pad, 4), jnp.float32),
          pltpu.SemaphoreType.DMA,
          pltpu.SemaphoreType.DMA,
      ],
  )
  def main(tbl, srcs, dsts, zeros, out,
           sidx, didx, sr0, dr0, sr1, dr1, contrib, accum, sem0, sem1):
    c = lax.axis_index("c")
    s = lax.axis_index("s")
    w = s * NC + c

    iot = _iota16()
    c3 = _c16(3)
    ones = jnp.full((LANES,), 1.0, jnp.float32)
    for q in range(CHUNK // LANES):
      plsc.store_scatter(contrib, [iot + q * LANES, c3], ones)

    pltpu.sync_copy(zeros.at[pl.ds(s * rows_t, rows_t)],
                    accum.at[pl.ds(s * rows_t, rows_t)])
    plsc.subcore_barrier()

    bufs = ((sr0, dr0, sem0), (sr1, dr1, sem1))
    base_sc = w * g_per_w

    def superchunk(g, carry):
      pltpu.sync_copy(srcs.at[base_sc + g], sidx)
      pltpu.sync_copy(dsts.at[base_sc + g], didx)
      handles = None
      for j in range(SCH):
        sbuf, dbuf, sem = bufs[j % 2]
        if j == 0:
          handles = (pltpu.async_copy(tbl.at[sidx.at[0]], sbuf, sem),
                     pltpu.async_copy(tbl.at[didx.at[0]], dbuf, sem))
        cur = handles
        if j + 1 < SCH:
          nbuf, ndbuf, nsem = bufs[(j + 1) % 2]
          handles = (pltpu.async_copy(tbl.at[sidx.at[j + 1]], nbuf, nsem),
                     pltpu.async_copy(tbl.at[didx.at[j + 1]], ndbuf, nsem))
        cur[0].wait()
        cur[1].wait()
        _edge_compute(sbuf, dbuf, contrib)
        pltpu.sync_copy(contrib, accum.at[didx.at[j]], add=True)
      return carry

    lax.fori_loop(0, g_per_w, superchunk, 0)

    plsc.subcore_barrier()
    pltpu.sync_copy(accum.at[pl.ds(s * rows_t, rows_t)],
                    out.at[c, pl.ds(s * rows_t, rows_t)])

  return main


def _make_finalize(n_pad):
  mesh = plsc.VectorSubcoreMesh(core_axis_name="c", subcore_axis_name="s")
  rows_w = n_pad // NW

  @functools.partial(
      pl.kernel,
      mesh=mesh,
      compiler_params=_SC_PARAMS,
      out_type=jax.ShapeDtypeStruct((n_pad, 3), jnp.float32),
      scratch_types=[
          pltpu.VMEM((rows_w, 4), jnp.float32),
          pltpu.VMEM((rows_w, 4), jnp.float32),
          pltpu.VMEM((rows_w,), jnp.float32),
          pltpu.VMEM((rows_w, 3), jnp.float32),
      ],
  )
  def fin(part, phi, fout, p0v, p1v, phv, outv):
    c = lax.axis_index("c")
    s = lax.axis_index("s")
    w = s * NC + c
    base = w * rows_w
    pltpu.sync_copy(part.at[0, pl.ds(base, rows_w)], p0v)
    pltpu.sync_copy(part.at[1, pl.ds(base, rows_w)], p1v)
    pltpu.sync_copy(phi.at[pl.ds(base, rows_w)], phv)

    iot = _iota16()
    c0 = _c16(0)
    c1 = _c16(1)
    c2 = _c16(2)
    c3 = _c16(3)

    def body(q, carry):
      row = iot + q * LANES
      nx = plsc.load_gather(p0v, [row, c0]) + plsc.load_gather(p1v, [row, c0])
      ny = plsc.load_gather(p0v, [row, c1]) + plsc.load_gather(p1v, [row, c1])
      nz = plsc.load_gather(p0v, [row, c2]) + plsc.load_gather(p1v, [row, c2])
      dg = plsc.load_gather(p0v, [row, c3]) + plsc.load_gather(p1v, [row, c3])
      ph = phv[pl.ds(q * LANES, LANES)]
      scale = jnp.float32(DSIGMA_DT) / jnp.maximum(dg, jnp.float32(1.0))
      scale = jnp.where(jnp.abs(ph) < jnp.float32(PHI_CUT), scale,
                        jnp.float32(0.0))
      plsc.store_scatter(outv, [row, c0], nx * scale)
      plsc.store_scatter(outv, [row, c1], ny * scale)
      plsc.store_scatter(outv, [row, c2], nz * scale)
      return carry

    lax.fori_loop(0, rows_w // LANES, body, 0)
    pltpu.sync_copy(outv, fout.at[pl.ds(base, rows_w)])

  return fin


def kernel(x, pos, edge_index):
  n = pos.shape[0]
  e = edge_index.shape[1]
  n_pad = ((n + 1 + 511) // 512) * 512
  edges_per_sup = NW * SCH * CHUNK
  g_per_w = (e + NW * SCH * CHUNK - 1) // (NW * SCH * CHUNK)
  e_pad = g_per_w * edges_per_sup

  t_col = x[:, 3]
  phi = x[:, 8]
  tbl = jnp.concatenate([pos, t_col[:, None]], axis=1)
  tbl = jnp.pad(tbl, ((0, n_pad - n), (0, 0)))
  phi_p = jnp.pad(phi, (0, n_pad - n))

  n_dummy = min(256, n_pad - n)
  pad_idx = (n + (jnp.arange(e_pad - e, dtype=jnp.int32) % n_dummy)
             ).astype(jnp.int32)
  srcp = jnp.concatenate([edge_index[0], pad_idx]).reshape(
      NW * g_per_w, SCH, CHUNK)
  dstp = jnp.concatenate([edge_index[1], pad_idx]).reshape(
      NW * g_per_w, SCH, CHUNK)
  zeros = jnp.zeros((n_pad, 4), jnp.float32)

  partial = _make_main(n_pad, g_per_w)(tbl, srcp, dstp, zeros)
  force = _make_finalize(n_pad)(partial, phi_p)
  return force[:n]


# speedup vs baseline: 125.7664x; 1.0238x over previous
import functools

import jax
import jax.numpy as jnp
from jax import lax
from jax.experimental import pallas as pl
from jax.experimental.pallas import tpu as pltpu
from jax.experimental.pallas import tpu_sc as plsc

DSIGMA_DT = -0.0001
PHI_CUT = 0.3
EPS = 1e-8

NC = 2
NS = 16
NW = NC * NS
CHUNK = 128
SCH = 8
LANES = 16
RW = 8

_SC_PARAMS = pltpu.CompilerParams(
    needs_layout_passes=False, use_tc_tiling_on_sc=False)


def _iota16():
  return lax.iota(jnp.int32, LANES)


def _c16(v, dtype=jnp.int32):
  return jnp.full((LANES,), v, dtype)


def _edge_compute(sbuf, dbuf, contrib):
  iot = _iota16()
  c0 = _c16(0)
  c1 = _c16(1)
  c2 = _c16(2)
  c3 = _c16(3)
  for q in range(CHUNK // LANES):
    row = iot + q * LANES
    sx = plsc.load_gather(sbuf, [row, c0])
    sy = plsc.load_gather(sbuf, [row, c1])
    sz = plsc.load_gather(sbuf, [row, c2])
    st = plsc.load_gather(sbuf, [row, c3])
    dx = plsc.load_gather(dbuf, [row, c0])
    dy = plsc.load_gather(dbuf, [row, c1])
    dz = plsc.load_gather(dbuf, [row, c2])
    dt = plsc.load_gather(dbuf, [row, c3])
    ex = sx - dx
    ey = sy - dy
    ez = sz - dz
    et = st - dt
    rr = ex * ex + ey * ey + ez * ez + jnp.float32(EPS)
    w = et / rr
    plsc.store_scatter(contrib, [row, c0], w * ex)
    plsc.store_scatter(contrib, [row, c1], w * ey)
    plsc.store_scatter(contrib, [row, c2], w * ez)


def _make_main(n_pad, g_per_w):
  mesh = plsc.VectorSubcoreMesh(core_axis_name="c", subcore_axis_name="s")
  rows_t = n_pad // NS

  @functools.partial(
      pl.kernel,
      mesh=mesh,
      compiler_params=_SC_PARAMS,
      out_type=jax.ShapeDtypeStruct((NC, n_pad, RW), jnp.float32),
      scratch_types=[
          pltpu.VMEM((SCH, CHUNK), jnp.int32),
          pltpu.VMEM((SCH, CHUNK), jnp.int32),
          pltpu.VMEM((CHUNK, RW), jnp.float32),
          pltpu.VMEM((CHUNK, RW), jnp.float32),
          pltpu.VMEM((CHUNK, RW), jnp.float32),
          pltpu.VMEM((CHUNK, RW), jnp.float32),
          pltpu.VMEM((CHUNK, RW), jnp.float32),
          pltpu.VMEM_SHARED((n_pad, RW), jnp.float32),
          pltpu.SemaphoreType.DMA,
          pltpu.SemaphoreType.DMA,
      ],
  )
  def main(tbl, srcs, dsts, zeros, out,
           sidx, didx, sr0, dr0, sr1, dr1, contrib, accum, sem0, sem1):
    c = lax.axis_index("c")
    s = lax.axis_index("s")
    w = s * NC + c

    iot = _iota16()
    ones = jnp.full((LANES,), 1.0, jnp.float32)
    zero16 = jnp.full((LANES,), 0.0, jnp.float32)
    for q in range(CHUNK // LANES):
      row = iot + q * LANES
      plsc.store_scatter(contrib, [row, _c16(3)], ones)
      for cc in range(4, RW):
        plsc.store_scatter(contrib, [row, _c16(cc)], zero16)

    pltpu.sync_copy(zeros.at[pl.ds(s * rows_t, rows_t)],
                    accum.at[pl.ds(s * rows_t, rows_t)])
    plsc.subcore_barrier()

    bufs = ((sr0, dr0, sem0), (sr1, dr1, sem1))
    base_sc = w * g_per_w

    def superchunk(g, carry):
      pltpu.sync_copy(srcs.at[base_sc + g], sidx)
      pltpu.sync_copy(dsts.at[base_sc + g], didx)
      handles = None
      for j in range(SCH):
        sbuf, dbuf, sem = bufs[j % 2]
        if j == 0:
          handles = (pltpu.async_copy(tbl.at[sidx.at[0]], sbuf, sem),
                     pltpu.async_copy(tbl.at[didx.at[0]], dbuf, sem))
        cur = handles
        if j + 1 < SCH:
          nbuf, ndbuf, nsem = bufs[(j + 1) % 2]
          handles = (pltpu.async_copy(tbl.at[sidx.at[j + 1]], nbuf, nsem),
                     pltpu.async_copy(tbl.at[didx.at[j + 1]], ndbuf, nsem))
        cur[0].wait()
        cur[1].wait()
        _edge_compute(sbuf, dbuf, contrib)
        pltpu.sync_copy(contrib, accum.at[didx.at[j]], add=True)
      return carry

    lax.fori_loop(0, g_per_w, superchunk, 0)

    plsc.subcore_barrier()
    pltpu.sync_copy(accum.at[pl.ds(s * rows_t, rows_t)],
                    out.at[c, pl.ds(s * rows_t, rows_t)])

  return main


def _make_finalize(n_pad):
  mesh = plsc.VectorSubcoreMesh(core_axis_name="c", subcore_axis_name="s")
  rows_w = n_pad // NW

  @functools.partial(
      pl.kernel,
      mesh=mesh,
      compiler_params=_SC_PARAMS,
      out_type=jax.ShapeDtypeStruct((n_pad, RW), jnp.float32),
      scratch_types=[
          pltpu.VMEM((rows_w, RW), jnp.float32),
          pltpu.VMEM((rows_w, RW), jnp.float32),
          pltpu.VMEM((rows_w,), jnp.float32),
          pltpu.VMEM((rows_w, RW), jnp.float32),
      ],
  )
  def fin(part, phi, fout, p0v, p1v, phv, outv):
    c = lax.axis_index("c")
    s = lax.axis_index("s")
    w = s * NC + c
    base = w * rows_w
    pltpu.sync_copy(part.at[0, pl.ds(base, rows_w)], p0v)
    pltpu.sync_copy(part.at[1, pl.ds(base, rows_w)], p1v)
    pltpu.sync_copy(phi.at[pl.ds(base, rows_w)], phv)

    iot = _iota16()
    c0 = _c16(0)
    c1 = _c16(1)
    c2 = _c16(2)
    c3 = _c16(3)

    def body(q, carry):
      row = iot + q * LANES
      nx = plsc.load_gather(p0v, [row, c0]) + plsc.load_gather(p1v, [row, c0])
      ny = plsc.load_gather(p0v, [row, c1]) + plsc.load_gather(p1v, [row, c1])
      nz = plsc.load_gather(p0v, [row, c2]) + plsc.load_gather(p1v, [row, c2])
      dg = plsc.load_gather(p0v, [row, c3]) + plsc.load_gather(p1v, [row, c3])
      ph = phv[pl.ds(q * LANES, LANES)]
      scale = jnp.float32(DSIGMA_DT) / jnp.maximum(dg, jnp.float32(1.0))
      scale = jnp.where(jnp.abs(ph) < jnp.float32(PHI_CUT), scale,
                        jnp.float32(0.0))
      plsc.store_scatter(outv, [row, c0], nx * scale)
      plsc.store_scatter(outv, [row, c1], ny * scale)
      plsc.store_scatter(outv, [row, c2], nz * scale)
      return carry

    lax.fori_loop(0, rows_w // LANES, body, 0)
    pltpu.sync_copy(outv, fout.at[pl.ds(base, rows_w)])

  return fin


def kernel(x, pos, edge_index):
  n = pos.shape[0]
  e = edge_index.shape[1]
  n_pad = ((n + 1 + 511) // 512) * 512
  edges_per_sup = NW * SCH * CHUNK
  g_per_w = (e + edges_per_sup - 1) // edges_per_sup
  e_pad = g_per_w * edges_per_sup

  t_col = x[:, 3]
  phi = x[:, 8]
  tbl = jnp.concatenate([pos, t_col[:, None]], axis=1)
  tbl = jnp.pad(tbl, ((0, n_pad - n), (0, RW - 4)))
  phi_p = jnp.pad(phi, (0, n_pad - n))

  n_dummy = min(256, n_pad - n)
  pad_idx = (n + (jnp.arange(e_pad - e, dtype=jnp.int32) % n_dummy)
             ).astype(jnp.int32)
  srcp = jnp.concatenate([edge_index[0], pad_idx]).reshape(
      NW * g_per_w, SCH, CHUNK)
  dstp = jnp.concatenate([edge_index[1], pad_idx]).reshape(
      NW * g_per_w, SCH, CHUNK)
  zeros = jnp.zeros((n_pad, RW), jnp.float32)

  partial = _make_main(n_pad, g_per_w)(tbl, srcp, dstp, zeros)
  force = _make_finalize(n_pad)(partial, phi_p)
  return force[:n, :3]


# speedup vs baseline: 147.1763x; 1.1702x over previous
import functools

import jax
import jax.numpy as jnp
from jax import lax
from jax.experimental import pallas as pl
from jax.experimental.pallas import tpu as pltpu
from jax.experimental.pallas import tpu_sc as plsc

DSIGMA_DT = -0.0001
PHI_CUT = 0.3
EPS = 1e-8

NC = 2
NS = 16
NW = NC * NS
CHUNK = 128
SCH = 8
LANES = 16
RW = 8

_SC_PARAMS = pltpu.CompilerParams(
    needs_layout_passes=False, use_tc_tiling_on_sc=False)


def _iota16():
  return lax.iota(jnp.int32, LANES)


def _c16(v, dtype=jnp.int32):
  return jnp.full((LANES,), v, dtype)


def _edge_compute(sbuf, dbuf, contrib):
  iot = _iota16()
  c0 = _c16(0)
  c1 = _c16(1)
  c2 = _c16(2)
  c3 = _c16(3)
  for q in range(CHUNK // LANES):
    row = iot + q * LANES
    sx = plsc.load_gather(sbuf, [row, c0])
    sy = plsc.load_gather(sbuf, [row, c1])
    sz = plsc.load_gather(sbuf, [row, c2])
    st = plsc.load_gather(sbuf, [row, c3])
    dx = plsc.load_gather(dbuf, [row, c0])
    dy = plsc.load_gather(dbuf, [row, c1])
    dz = plsc.load_gather(dbuf, [row, c2])
    dt = plsc.load_gather(dbuf, [row, c3])
    ex = sx - dx
    ey = sy - dy
    ez = sz - dz
    et = st - dt
    rr = ex * ex + ey * ey + ez * ez + jnp.float32(EPS)
    w = et / rr
    plsc.store_scatter(contrib, [row, c0], w * ex)
    plsc.store_scatter(contrib, [row, c1], w * ey)
    plsc.store_scatter(contrib, [row, c2], w * ez)


def _make_main(n_pad, g_per_w):
  mesh = plsc.VectorSubcoreMesh(core_axis_name="c", subcore_axis_name="s")
  rows_t = n_pad // NS

  @functools.partial(
      pl.kernel,
      mesh=mesh,
      compiler_params=_SC_PARAMS,
      out_type=jax.ShapeDtypeStruct((NC, n_pad, RW), jnp.float32),
      scratch_types=[
          pltpu.VMEM((SCH, CHUNK), jnp.int32),
          pltpu.VMEM((SCH, CHUNK), jnp.int32),
          pltpu.VMEM((SCH, CHUNK), jnp.int32),
          pltpu.VMEM((SCH, CHUNK), jnp.int32),
          pltpu.VMEM((CHUNK, RW), jnp.float32),
          pltpu.VMEM((CHUNK, RW), jnp.float32),
          pltpu.VMEM((CHUNK, RW), jnp.float32),
          pltpu.VMEM((CHUNK, RW), jnp.float32),
          pltpu.VMEM((CHUNK, RW), jnp.float32),
          pltpu.VMEM((CHUNK, RW), jnp.float32),
          pltpu.VMEM_SHARED((n_pad, RW), jnp.float32),
          pltpu.SemaphoreType.DMA,
          pltpu.SemaphoreType.DMA,
          pltpu.SemaphoreType.DMA,
          pltpu.SemaphoreType.DMA,
          pltpu.SemaphoreType.DMA,
          pltpu.SemaphoreType.DMA,
      ],
  )
  def main(tbl, srcs, dsts, zeros, out,
           sxa, dxa, sxb, dxb, sr0, dr0, sr1, dr1, cb0, cb1,
           accum, semg0, semg1, sems0, sems1, semia, semib):
    c = lax.axis_index("c")
    s = lax.axis_index("s")
    w = s * NC + c

    iot = _iota16()
    ones = jnp.full((LANES,), 1.0, jnp.float32)
    zero16 = jnp.full((LANES,), 0.0, jnp.float32)
    for q in range(CHUNK // LANES):
      row = iot + q * LANES
      for contrib in (cb0, cb1):
        plsc.store_scatter(contrib, [row, _c16(3)], ones)
        for cc in range(4, RW):
          plsc.store_scatter(contrib, [row, _c16(cc)], zero16)

    pltpu.sync_copy(zeros.at[pl.ds(s * rows_t, rows_t)],
                    accum.at[pl.ds(s * rows_t, rows_t)])
    plsc.subcore_barrier()

    bufs = ((sr0, dr0, semg0), (sr1, dr1, semg1))
    cbufs = ((cb0, sems0), (cb1, sems1))
    base_sc = w * g_per_w
    last_row = NW * g_per_w - 1

    def fire_idx(g_row, sx, dx, sem):
      pltpu.async_copy(srcs.at[g_row], sx, sem)
      pltpu.async_copy(dsts.at[g_row], dx, sem)

    def wait_idx(sx, dx, sem):
      pltpu.make_async_copy(srcs.at[0], sx, sem).wait()
      pltpu.make_async_copy(dsts.at[0], dx, sem).wait()

    def do_superchunk(sidx, didx):
      handles = None
      sh = [None, None]
      for j in range(SCH):
        sbuf, dbuf, sem = bufs[j % 2]
        cbuf, csem = cbufs[j % 2]
        if j == 0:
          handles = (pltpu.async_copy(tbl.at[sidx.at[0]], sbuf, sem),
                     pltpu.async_copy(tbl.at[didx.at[0]], dbuf, sem))
        cur = handles
        if j + 1 < SCH:
          nbuf, ndbuf, nsem = bufs[(j + 1) % 2]
          handles = (pltpu.async_copy(tbl.at[sidx.at[j + 1]], nbuf, nsem),
                     pltpu.async_copy(tbl.at[didx.at[j + 1]], ndbuf, nsem))
        cur[0].wait()
        cur[1].wait()
        if sh[j % 2] is not None:
          sh[j % 2].wait()
        _edge_compute(sbuf, dbuf, cbuf)
        sh[j % 2] = pltpu.async_copy(cbuf, accum.at[didx.at[j]], csem,
                                     add=True)
      sh[0].wait()
      sh[1].wait()

    fire_idx(base_sc, sxa, dxa, semia)
    fire_idx(base_sc + 1, sxb, dxb, semib)

    def pair(t, carry):
      g = 2 * t
      wait_idx(sxa, dxa, semia)
      do_superchunk(sxa, dxa)
      fire_idx(jnp.minimum(base_sc + g + 2, last_row), sxa, dxa, semia)
      wait_idx(sxb, dxb, semib)
      do_superchunk(sxb, dxb)
      fire_idx(jnp.minimum(base_sc + g + 3, last_row), sxb, dxb, semib)
      return carry

    lax.fori_loop(0, g_per_w // 2, pair, 0)
    wait_idx(sxa, dxa, semia)
    wait_idx(sxb, dxb, semib)

    plsc.subcore_barrier()
    pltpu.sync_copy(accum.at[pl.ds(s * rows_t, rows_t)],
                    out.at[c, pl.ds(s * rows_t, rows_t)])

  return main


def _make_finalize(n_pad):
  mesh = plsc.VectorSubcoreMesh(core_axis_name="c", subcore_axis_name="s")
  rows_w = n_pad // NW

  @functools.partial(
      pl.kernel,
      mesh=mesh,
      compiler_params=_SC_PARAMS,
      out_type=jax.ShapeDtypeStruct((n_pad, RW), jnp.float32),
      scratch_types=[
          pltpu.VMEM((rows_w, RW), jnp.float32),
          pltpu.VMEM((rows_w, RW), jnp.float32),
          pltpu.VMEM((rows_w,), jnp.float32),
          pltpu.VMEM((rows_w, RW), jnp.float32),
      ],
  )
  def fin(part, phi, fout, p0v, p1v, phv, outv):
    c = lax.axis_index("c")
    s = lax.axis_index("s")
    w = s * NC + c
    base = w * rows_w
    pltpu.sync_copy(part.at[0, pl.ds(base, rows_w)], p0v)
    pltpu.sync_copy(part.at[1, pl.ds(base, rows_w)], p1v)
    pltpu.sync_copy(phi.at[pl.ds(base, rows_w)], phv)

    iot = _iota16()
    c0 = _c16(0)
    c1 = _c16(1)
    c2 = _c16(2)
    c3 = _c16(3)

    def body(q, carry):
      row = iot + q * LANES
      nx = plsc.load_gather(p0v, [row, c0]) + plsc.load_gather(p1v, [row, c0])
      ny = plsc.load_gather(p0v, [row, c1]) + plsc.load_gather(p1v, [row, c1])
      nz = plsc.load_gather(p0v, [row, c2]) + plsc.load_gather(p1v, [row, c2])
      dg = plsc.load_gather(p0v, [row, c3]) + plsc.load_gather(p1v, [row, c3])
      ph = phv[pl.ds(q * LANES, LANES)]
      scale = jnp.float32(DSIGMA_DT) / jnp.maximum(dg, jnp.float32(1.0))
      scale = jnp.where(jnp.abs(ph) < jnp.float32(PHI_CUT), scale,
                        jnp.float32(0.0))
      plsc.store_scatter(outv, [row, c0], nx * scale)
      plsc.store_scatter(outv, [row, c1], ny * scale)
      plsc.store_scatter(outv, [row, c2], nz * scale)
      return carry

    lax.fori_loop(0, rows_w // LANES, body, 0)
    pltpu.sync_copy(outv, fout.at[pl.ds(base, rows_w)])

  return fin


def kernel(x, pos, edge_index):
  n = pos.shape[0]
  e = edge_index.shape[1]
  n_pad = ((n + 1 + 511) // 512) * 512
  edges_per_sup = NW * SCH * CHUNK
  g_per_w = (e + 2 * edges_per_sup - 1) // (2 * edges_per_sup) * 2
  e_pad = g_per_w * edges_per_sup

  t_col = x[:, 3]
  phi = x[:, 8]
  tbl = jnp.concatenate([pos, t_col[:, None]], axis=1)
  tbl = jnp.pad(tbl, ((0, n_pad - n), (0, RW - 4)))
  phi_p = jnp.pad(phi, (0, n_pad - n))

  n_dummy = min(256, n_pad - n)
  pad_idx = (n + (jnp.arange(e_pad - e, dtype=jnp.int32) % n_dummy)
             ).astype(jnp.int32)
  srcp = jnp.concatenate([edge_index[0], pad_idx]).reshape(
      NW * g_per_w, SCH, CHUNK)
  dstp = jnp.concatenate([edge_index[1], pad_idx]).reshape(
      NW * g_per_w, SCH, CHUNK)
  zeros = jnp.zeros((n_pad, RW), jnp.float32)

  partial = _make_main(n_pad, g_per_w)(tbl, srcp, dstp, zeros)
  force = _make_finalize(n_pad)(partial, phi_p)
  return force[:n, :3]


# speedup vs baseline: 218.7075x; 1.4860x over previous
import functools

import jax
import jax.numpy as jnp
from jax import lax
from jax.experimental import pallas as pl
from jax.experimental.pallas import tpu as pltpu
from jax.experimental.pallas import tpu_sc as plsc

DSIGMA_DT = -0.0001
PHI_CUT = 0.3
EPS = 1e-8

NC = 2
NS = 16
NW = NC * NS
CHUNK = 128
SCH = 8
LANES = 16
RW = 8

_SC_PARAMS = pltpu.CompilerParams(
    needs_layout_passes=False, use_tc_tiling_on_sc=False)


def _iota16():
  return lax.iota(jnp.int32, LANES)


def _c16(v, dtype=jnp.int32):
  return jnp.full((LANES,), v, dtype)


def _edge_compute(sbuf, dbuf, contrib):
  iot = _iota16()
  c0 = _c16(0)
  c1 = _c16(1)
  c2 = _c16(2)
  c3 = _c16(3)
  for q in range(CHUNK // LANES):
    row = iot + q * LANES
    sx = plsc.load_gather(sbuf, [row, c0])
    sy = plsc.load_gather(sbuf, [row, c1])
    sz = plsc.load_gather(sbuf, [row, c2])
    st = plsc.load_gather(sbuf, [row, c3])
    dx = plsc.load_gather(dbuf, [row, c0])
    dy = plsc.load_gather(dbuf, [row, c1])
    dz = plsc.load_gather(dbuf, [row, c2])
    dt = plsc.load_gather(dbuf, [row, c3])
    ex = sx - dx
    ey = sy - dy
    ez = sz - dz
    et = st - dt
    rr = ex * ex + ey * ey + ez * ez + jnp.float32(EPS)
    w = et / rr
    plsc.store_scatter(contrib, [row, c0], w * ex)
    plsc.store_scatter(contrib, [row, c1], w * ey)
    plsc.store_scatter(contrib, [row, c2], w * ez)


def _make_main(n_pad, g_per_w):
  mesh = plsc.VectorSubcoreMesh(core_axis_name="c", subcore_axis_name="s")
  rows_t = n_pad // NS

  @functools.partial(
      pl.kernel,
      mesh=mesh,
      compiler_params=_SC_PARAMS,
      out_type=jax.ShapeDtypeStruct((NC, n_pad, RW), jnp.float32),
      scratch_types=[
          pltpu.VMEM((SCH, CHUNK), jnp.int32),
          pltpu.VMEM((SCH, CHUNK), jnp.int32),
          pltpu.VMEM((SCH, CHUNK), jnp.int32),
          pltpu.VMEM((SCH, CHUNK), jnp.int32),
          pltpu.VMEM((CHUNK, RW), jnp.float32),
          pltpu.VMEM((CHUNK, RW), jnp.float32),
          pltpu.VMEM((CHUNK, RW), jnp.float32),
          pltpu.VMEM((CHUNK, RW), jnp.float32),
          pltpu.VMEM((CHUNK, RW), jnp.float32),
          pltpu.VMEM((CHUNK, RW), jnp.float32),
          pltpu.VMEM_SHARED((n_pad, RW), jnp.float32),
          pltpu.VMEM_SHARED((n_pad, RW), jnp.float32),
          pltpu.SemaphoreType.DMA,
          pltpu.SemaphoreType.DMA,
          pltpu.SemaphoreType.DMA,
          pltpu.SemaphoreType.DMA,
          pltpu.SemaphoreType.DMA,
          pltpu.SemaphoreType.DMA,
      ],
  )
  def main(tbl, srcs, dsts, zeros, out,
           sxa, dxa, sxb, dxb, sr0, dr0, sr1, dr1, cb0, cb1,
           accum, tbl_sh, semg0, semg1, sems0, sems1, semia, semib):
    c = lax.axis_index("c")
    s = lax.axis_index("s")
    w = s * NC + c

    iot = _iota16()
    ones = jnp.full((LANES,), 1.0, jnp.float32)
    zero16 = jnp.full((LANES,), 0.0, jnp.float32)
    for q in range(CHUNK // LANES):
      row = iot + q * LANES
      for contrib in (cb0, cb1):
        plsc.store_scatter(contrib, [row, _c16(3)], ones)
        for cc in range(4, RW):
          plsc.store_scatter(contrib, [row, _c16(cc)], zero16)

    pltpu.sync_copy(zeros.at[pl.ds(s * rows_t, rows_t)],
                    accum.at[pl.ds(s * rows_t, rows_t)])
    pltpu.sync_copy(tbl.at[pl.ds(s * rows_t, rows_t)],
                    tbl_sh.at[pl.ds(s * rows_t, rows_t)])
    plsc.subcore_barrier()

    bufs = ((sr0, dr0, semg0), (sr1, dr1, semg1))
    cbufs = ((cb0, sems0), (cb1, sems1))
    base_sc = w * g_per_w
    last_row = NW * g_per_w - 1

    def fire_idx(g_row, sx, dx, sem):
      pltpu.async_copy(srcs.at[g_row], sx, sem)
      pltpu.async_copy(dsts.at[g_row], dx, sem)

    def wait_idx(sx, dx, sem):
      pltpu.make_async_copy(srcs.at[0], sx, sem).wait()
      pltpu.make_async_copy(dsts.at[0], dx, sem).wait()

    def do_superchunk(sidx, didx):
      handles = None
      sh = [None, None]
      for j in range(SCH):
        sbuf, dbuf, sem = bufs[j % 2]
        cbuf, csem = cbufs[j % 2]
        if j == 0:
          handles = (pltpu.async_copy(tbl_sh.at[sidx.at[0]], sbuf, sem),
                     pltpu.async_copy(tbl_sh.at[didx.at[0]], dbuf, sem))
        cur = handles
        if j + 1 < SCH:
          nbuf, ndbuf, nsem = bufs[(j + 1) % 2]
          handles = (pltpu.async_copy(tbl_sh.at[sidx.at[j + 1]], nbuf, nsem),
                     pltpu.async_copy(tbl_sh.at[didx.at[j + 1]], ndbuf, nsem))
        cur[0].wait()
        cur[1].wait()
        if sh[j % 2] is not None:
          sh[j % 2].wait()
        _edge_compute(sbuf, dbuf, cbuf)
        sh[j % 2] = pltpu.async_copy(cbuf, accum.at[didx.at[j]], csem,
                                     add=True)
      sh[0].wait()
      sh[1].wait()

    fire_idx(base_sc, sxa, dxa, semia)
    fire_idx(base_sc + 1, sxb, dxb, semib)

    def pair(t, carry):
      g = 2 * t
      wait_idx(sxa, dxa, semia)
      do_superchunk(sxa, dxa)
      fire_idx(jnp.minimum(base_sc + g + 2, last_row), sxa, dxa, semia)
      wait_idx(sxb, dxb, semib)
      do_superchunk(sxb, dxb)
      fire_idx(jnp.minimum(base_sc + g + 3, last_row), sxb, dxb, semib)
      return carry

    lax.fori_loop(0, g_per_w // 2, pair, 0)
    wait_idx(sxa, dxa, semia)
    wait_idx(sxb, dxb, semib)

    plsc.subcore_barrier()
    pltpu.sync_copy(accum.at[pl.ds(s * rows_t, rows_t)],
                    out.at[c, pl.ds(s * rows_t, rows_t)])

  return main


def _make_finalize(n_pad):
  mesh = plsc.VectorSubcoreMesh(core_axis_name="c", subcore_axis_name="s")
  rows_w = n_pad // NW

  @functools.partial(
      pl.kernel,
      mesh=mesh,
      compiler_params=_SC_PARAMS,
      out_type=jax.ShapeDtypeStruct((n_pad, RW), jnp.float32),
      scratch_types=[
          pltpu.VMEM((rows_w, RW), jnp.float32),
          pltpu.VMEM((rows_w, RW), jnp.float32),
          pltpu.VMEM((rows_w,), jnp.float32),
          pltpu.VMEM((rows_w, RW), jnp.float32),
      ],
  )
  def fin(part, phi, fout, p0v, p1v, phv, outv):
    c = lax.axis_index("c")
    s = lax.axis_index("s")
    w = s * NC + c
    base = w * rows_w
    pltpu.sync_copy(part.at[0, pl.ds(base, rows_w)], p0v)
    pltpu.sync_copy(part.at[1, pl.ds(base, rows_w)], p1v)
    pltpu.sync_copy(phi.at[pl.ds(base, rows_w)], phv)

    iot = _iota16()
    c0 = _c16(0)
    c1 = _c16(1)
    c2 = _c16(2)
    c3 = _c16(3)

    def body(q, carry):
      row = iot + q * LANES
      nx = plsc.load_gather(p0v, [row, c0]) + plsc.load_gather(p1v, [row, c0])
      ny = plsc.load_gather(p0v, [row, c1]) + plsc.load_gather(p1v, [row, c1])
      nz = plsc.load_gather(p0v, [row, c2]) + plsc.load_gather(p1v, [row, c2])
      dg = plsc.load_gather(p0v, [row, c3]) + plsc.load_gather(p1v, [row, c3])
      ph = phv[pl.ds(q * LANES, LANES)]
      scale = jnp.float32(DSIGMA_DT) / jnp.maximum(dg, jnp.float32(1.0))
      scale = jnp.where(jnp.abs(ph) < jnp.float32(PHI_CUT), scale,
                        jnp.float32(0.0))
      plsc.store_scatter(outv, [row, c0], nx * scale)
      plsc.store_scatter(outv, [row, c1], ny * scale)
      plsc.store_scatter(outv, [row, c2], nz * scale)
      return carry

    lax.fori_loop(0, rows_w // LANES, body, 0)
    pltpu.sync_copy(outv, fout.at[pl.ds(base, rows_w)])

  return fin


def kernel(x, pos, edge_index):
  n = pos.shape[0]
  e = edge_index.shape[1]
  n_pad = ((n + 1 + 511) // 512) * 512
  edges_per_sup = NW * SCH * CHUNK
  g_per_w = (e + 2 * edges_per_sup - 1) // (2 * edges_per_sup) * 2
  e_pad = g_per_w * edges_per_sup

  t_col = x[:, 3]
  phi = x[:, 8]
  tbl = jnp.concatenate([pos, t_col[:, None]], axis=1)
  tbl = jnp.pad(tbl, ((0, n_pad - n), (0, RW - 4)))
  phi_p = jnp.pad(phi, (0, n_pad - n))

  n_dummy = min(256, n_pad - n)
  pad_idx = (n + (jnp.arange(e_pad - e, dtype=jnp.int32) % n_dummy)
             ).astype(jnp.int32)
  srcp = jnp.concatenate([edge_index[0], pad_idx]).reshape(
      NW * g_per_w, SCH, CHUNK)
  dstp = jnp.concatenate([edge_index[1], pad_idx]).reshape(
      NW * g_per_w, SCH, CHUNK)
  zeros = jnp.zeros((n_pad, RW), jnp.float32)

  partial = _make_main(n_pad, g_per_w)(tbl, srcp, dstp, zeros)
  force = _make_finalize(n_pad)(partial, phi_p)
  return force[:n, :3]
